# baseline (speedup 1.0000x reference)
import functools

import jax
import jax.numpy as jnp
from jax import lax
from jax.experimental import pallas as pl
from jax.experimental.pallas import tpu as pltpu
from jax.experimental.pallas import tpu_sc as plsc

N = 65536
D = 64
K = 32
BOUND = 3.0
MIN_BW = 1e-3
MIN_BH = 1e-3
MIN_D = 1e-3
MIN_L = 0.025
EPS = 1e-6
LN2 = 0.6931471805599453
G = 1024

ROW = 2 * K + 1
GP = G // 4
GROW = GP + 1
OFF_M = 0
OFF_IS = OFF_M + D
OFF_CLO = OFF_IS + D
OFF_CSC = OFF_CLO + D
OFF_T64 = OFF_CSC + D
OFF_AF = OFF_T64 + D * ROW
OFF_BF = OFF_AF + D * ROW
OFF_A = OFF_BF + D * ROW
OFF_B = OFF_A + D * ROW
OFF_L = OFF_B + D * ROW
OFF_C0 = OFF_L + D * ROW
TOTAL = OFF_C0 + 16

NCORES = 2
NSUB = 16
NW = NCORES * NSUB
SPW = N // NW
CH = 1024
NG = CH // 16


def _prep_body(means_ref, stds_ref, uw_ref, uh_ref, udl_ref, udr_ref, ul_ref,
               is_ref, clo_ref, csc_ref, thr_ref, yc_ref,
               af0_ref, af1_ref, bf0_ref, bf1_ref,
               a0_ref, a1_ref, b0_ref, b1_ref, l0_ref, l1_ref, c0_ref,
               grid_ref):
    j = lax.broadcasted_iota(jnp.int32, (D, K), 1)
    kk = lax.broadcasted_iota(jnp.int32, (K, K), 0)
    jj = lax.broadcasted_iota(jnp.int32, (K, K), 1)
    tri_incl = (kk <= jj).astype(jnp.float32)
    tri_excl = (kk < jj).astype(jnp.float32)

    def softmax(x):
        e = jnp.exp(x - jnp.max(x, axis=-1, keepdims=True))
        return e / jnp.sum(e, axis=-1, keepdims=True)

    def softplus(x):
        return jnp.maximum(x, 0.0) + jnp.log(1.0 + jnp.exp(-jnp.abs(x)))

    w = softmax(uw_ref[...])
    h = softmax(uh_ref[...])
    wbins = MIN_BW + (1.0 - MIN_BW * K) * w
    hbins = MIN_BH + (1.0 - MIN_BH * K) * h

    def mm(x, t):
        return lax.dot_general(x, t, (((1,), (0,)), ((), ())),
                               precision=lax.Precision.HIGHEST,
                               preferred_element_type=jnp.float32)

    csw_i = mm(wbins, tri_incl)
    csw_e = mm(wbins, tri_excl)
    csh_i = mm(hbins, tri_incl)
    csh_e = mm(hbins, tri_excl)
    cwk = 2.0 * BOUND * csw_e - BOUND
    cw_next = jnp.where(j == K - 1, BOUND, 2.0 * BOUND * csw_i - BOUND)
    chk = 2.0 * BOUND * csh_e - BOUND
    ch_next = jnp.where(j == K - 1, BOUND, 2.0 * BOUND * csh_i - BOUND)
    widths = cw_next - cwk
    heights = ch_next - chk
    delta = heights / widths

    lam = MIN_L + (1.0 - 2.0 * MIN_L) / (1.0 + jnp.exp(-ul_ref[...]))
    dk = jnp.where(j == 0, 1.0 - MIN_D, MIN_D + softplus(udl_ref[...]))
    dk1 = jnp.where(j == K - 1, 1.0 - MIN_D, MIN_D + softplus(udr_ref[...]))
    wb = jnp.sqrt(dk / dk1)
    wc = (lam * dk + (1.0 - lam) * wb * dk1) / delta
    ya = chk
    yb = ch_next
    yc = ((1.0 - lam) * ya + lam * wb * yb) / ((1.0 - lam) + lam * wb)

    A0 = -lam
    B0 = lam * ya
    a0 = wc - 1.0
    b0 = ya - wc * yc
    A1 = wc - lam * wb
    B1 = lam * wb * yb - wc * yc
    a1 = wc - wb
    b1 = wb * yb - wc * yc
    af0 = A0 * widths + cwk * a0
    bf0 = B0 * widths + cwk * b0
    af1 = A1 * widths + cwk * a1
    bf1 = B1 * widths + cwk * b1
    thr = jnp.where(j == K - 1, 1e30, ch_next + EPS)

    mm_ = means_ref[...]
    s10 = 10.0 * stds_ref[...]
    is_ = 1.0 / s10
    mis = mm_ * is_
    af0_ref[...] = af0 * is_
    bf0_ref[...] = bf0 - af0 * mis
    af1_ref[...] = af1 * is_
    bf1_ref[...] = bf1 - af1 * mis
    a0_ref[...] = a0 * is_
    b0_ref[...] = b0 - a0 * mis
    a1_ref[...] = a1 * is_
    b1_ref[...] = b1 - a1 * mis
    l0_ref[...] = jnp.log(wc * lam * (yc - ya) * widths)
    l1_ref[...] = jnp.log(wb * wc * (1.0 - lam) * (yb - yc) * widths)
    ycx = mm_ + yc * s10
    thrx = mm_ + thr * s10
    ycb = lax.bitcast_convert_type(ycx, jnp.int32)
    ycb = jnp.where(ycx >= 0, ycb + 1, ycb - 1)
    ycx = jnp.where(ycx == 0.0,
                    jnp.float32(1.1754944e-38),
                    lax.bitcast_convert_type(ycb, jnp.float32))
    yc_ref[...] = ycx
    thr_ref[...] = thrx
    is_ref[...] = is_
    clo = mm_ - BOUND * s10
    clo_ref[...] = clo
    csc_ref[...] = is_ * (G / 6.0)
    c0 = (-(D / 2) * jnp.log(2.0 * jnp.pi) - jnp.sum(jnp.log(s10)))
    c0_ref[...] = jnp.broadcast_to(c0, (1, 1))

    cellw = (6.0 / G) * s10
    base_i = lax.broadcasted_iota(jnp.int32, (D, GP), 1) * 4
    packed = jnp.zeros((D, GP), jnp.int32)
    for p in range(4):
        left = clo + (base_i + p).astype(jnp.float32) * cellw
        acc = jnp.zeros((D, GP), jnp.int32)
        for m in range(2 * K - 1):
            if m % 2 == 0:
                t = ycx[:, m // 2:m // 2 + 1]
            else:
                t = thrx[:, (m - 1) // 2:(m - 1) // 2 + 1]
            acc = acc + (left >= t).astype(jnp.int32)
        packed = packed | (jnp.minimum(acc, 2 * K - 4) << (8 * p))
    grid_ref[...] = packed


_DK = jax.ShapeDtypeStruct((D, K), jnp.float32)
_prep = pl.pallas_call(
    _prep_body,
    out_shape=[jax.ShapeDtypeStruct((D, 1), jnp.float32)] * 3 + [_DK] * 12
              + [jax.ShapeDtypeStruct((1, 1), jnp.float32),
                 jax.ShapeDtypeStruct((D, GP), jnp.int32)],
)


def _sc_body(x_hbm, tbl_hbm, grid_hbm, out_hbm, tbl_v, grid_v, x_v, out_v):
    wid = lax.axis_index("s") * NCORES + lax.axis_index("c")
    base = wid * SPW
    pltpu.sync_copy(tbl_hbm, tbl_v)
    pltpu.sync_copy(grid_hbm, grid_v)
    lane = lax.iota(jnp.int32, 16)
    c0v = tbl_v[pl.ds(OFF_C0, 16)]

    for chunk in range(SPW // CH):
        pltpu.sync_copy(
            x_hbm.at[pl.ds((base + chunk * CH) * D, CH * D)], x_v)

        def init_body(g, carry):
            out_v[pl.ds(chunk * CH + g * 16, 16)] = c0v
            return carry

        lax.fori_loop(0, NG, init_body, 0)

        def d_body(dd, carry):
            dvec = (dd + lane) & (D - 1)
            mv = plsc.load_gather(tbl_v, [dvec + OFF_M])
            isv = plsc.load_gather(tbl_v, [dvec + OFF_IS])
            clov = plsc.load_gather(tbl_v, [dvec + OFF_CLO])
            cscv = plsc.load_gather(tbl_v, [dvec + OFF_CSC])
            xib = lane * D + dvec
            rowv = dvec * ROW
            gbase = dvec * GROW
            tbase = rowv + OFF_T64
            caf = rowv + OFF_AF
            cbf = rowv + OFF_BF
            ca = rowv + OFF_A
            cb = rowv + OFF_B
            cl = rowv + OFF_L

            @plsc.parallel_loop(0, NG, step=1, unroll=6)
            def g_loop(g):
                xv = plsc.load_gather(x_v, [xib + g * (16 * D)])
                z1 = (xv - mv) * isv
                inside = (z1 >= -BOUND) & (z1 <= BOUND)
                cell = jnp.minimum(jnp.maximum(
                    ((xv - clov) * cscv).astype(jnp.int32), 0), G - 1)
                cw2 = plsc.load_gather(grid_v, [gbase + (cell >> 2)])
                lo = (cw2 >> ((cell & 3) << 3)) & 0xFF
                q = lo
                for i in range(3):
                    t = plsc.load_gather(tbl_v, [tbase + (lo + i)])
                    q = q + (xv >= t).astype(jnp.int32)
                af = plsc.load_gather(tbl_v, [q + caf])
                bf = plsc.load_gather(tbl_v, [q + cbf])
                av = plsc.load_gather(tbl_v, [q + ca])
                bv = plsc.load_gather(tbl_v, [q + cb])
                lv = plsc.load_gather(tbl_v, [q + cl])
                den = av * xv + bv
                z0 = (af * xv + bf) / den
                ad = jnp.abs(den)
                bits = plsc.bitcast(ad, jnp.int32)
                e = jnp.right_shift(bits, 23) - 127
                mant = plsc.bitcast(
                    (bits & 0x007FFFFF) | 0x3F800000, jnp.float32)
                t5 = mant - 1.0
                p = t5 * (0.99943129 + t5 * (-0.49130654 + t5 * (0.28768438
                         + t5 * (-0.13394622 + t5 * 0.03129158))))
                ln = e.astype(jnp.float32) * LN2 + p
                contrib = lv - 2.0 * ln - 0.5 * z0 * z0
                res = jnp.where(inside, contrib, -0.5 * z1 * z1)
                plsc.addupdate(out_v.at[pl.ds(chunk * CH + g * 16, 16)], res)

            return carry

        lax.fori_loop(0, D, d_body, 0)

    pltpu.sync_copy(out_v, out_hbm.at[pl.ds(base, SPW)])


_sc_spline = functools.partial(
    pl.kernel,
    out_type=jax.ShapeDtypeStruct((N,), jnp.float32),
    mesh=plsc.VectorSubcoreMesh(core_axis_name="c", subcore_axis_name="s"),
    compiler_params=pltpu.CompilerParams(needs_layout_passes=False),
    scratch_types=[
        pltpu.VMEM((TOTAL,), jnp.float32),
        pltpu.VMEM((D * GROW,), jnp.int32),
        pltpu.VMEM((CH * D,), jnp.float32),
        pltpu.VMEM((SPW,), jnp.float32),
    ],
)(_sc_body)


def kernel(data_samples, ds_means, ds_stds, unnormalized_widths,
           unnormalized_heights, unnormalized_derivatives,
           unnormalized_lambdas):
    means2 = ds_means.reshape(D, 1)
    stds2 = ds_stds.reshape(D, 1)
    udl = jnp.pad(unnormalized_derivatives, ((0, 0), (1, 0)))
    udr = jnp.pad(unnormalized_derivatives, ((0, 0), (0, 1)))
    (is_, clo, csc, thr, yc, af0, af1, bf0, bf1, a0, a1, b0, b1, l0, l1, c0,
     grid) = _prep(means2, stds2, unnormalized_widths, unnormalized_heights,
                   udl, udr, unnormalized_lambdas)

    def il(x0, x1):
        x = jnp.stack([x0, x1], axis=2).reshape(D, 2 * K)
        return jnp.pad(x, ((0, 0), (0, 1))).reshape(-1)

    tbl = jnp.concatenate([
        ds_means.reshape(-1), is_.reshape(-1), clo.reshape(-1),
        csc.reshape(-1), il(yc, thr),
        il(af0, af1), il(bf0, bf1), il(a0, a1), il(b0, b1), il(l0, l1),
        jnp.broadcast_to(c0.reshape(()), (16,)),
    ])
    gridp = jnp.pad(grid, ((0, 0), (0, 1))).reshape(-1)
    return _sc_spline(data_samples.reshape(-1), tbl, gridp)


# speedup vs baseline: 1.0895x; 1.0895x over previous
import functools

import jax
import jax.numpy as jnp
from jax import lax
from jax.experimental import pallas as pl
from jax.experimental.pallas import tpu as pltpu
from jax.experimental.pallas import tpu_sc as plsc

N = 65536
D = 64
K = 32
BOUND = 3.0
MIN_BW = 1e-3
MIN_BH = 1e-3
MIN_D = 1e-3
MIN_L = 0.025
EPS = 1e-6
LN2 = 0.6931471805599453
G = 1024

ROW = 2 * K + 1
GP = G // 4
GROW = GP + 1
OFF_M = 0
OFF_IS = OFF_M + D
OFF_CLO = OFF_IS + D
OFF_CSC = OFF_CLO + D
OFF_T64 = OFF_CSC + D
OFF_AF = OFF_T64 + D * ROW
OFF_BF = OFF_AF + D * ROW
OFF_A = OFF_BF + D * ROW
OFF_B = OFF_A + D * ROW
OFF_L = OFF_B + D * ROW
OFF_C0 = OFF_L + D * ROW
TOTAL = OFF_C0 + 16

NCORES = 2
NSUB = 16
NW = NCORES * NSUB
SPW = N // NW
CH = 512
NG = CH // 16


def _prep_body(means_ref, stds_ref, uw_ref, uh_ref, udl_ref, udr_ref, ul_ref,
               is_ref, clo_ref, csc_ref, thr_ref, yc_ref,
               af0_ref, af1_ref, bf0_ref, bf1_ref,
               a0_ref, a1_ref, b0_ref, b1_ref, l0_ref, l1_ref, c0_ref,
               grid_ref):
    j = lax.broadcasted_iota(jnp.int32, (D, K), 1)
    kk = lax.broadcasted_iota(jnp.int32, (K, K), 0)
    jj = lax.broadcasted_iota(jnp.int32, (K, K), 1)
    tri_incl = (kk <= jj).astype(jnp.float32)
    tri_excl = (kk < jj).astype(jnp.float32)

    def softmax(x):
        e = jnp.exp(x - jnp.max(x, axis=-1, keepdims=True))
        return e / jnp.sum(e, axis=-1, keepdims=True)

    def softplus(x):
        return jnp.maximum(x, 0.0) + jnp.log(1.0 + jnp.exp(-jnp.abs(x)))

    w = softmax(uw_ref[...])
    h = softmax(uh_ref[...])
    wbins = MIN_BW + (1.0 - MIN_BW * K) * w
    hbins = MIN_BH + (1.0 - MIN_BH * K) * h

    def mm(x, t):
        return lax.dot_general(x, t, (((1,), (0,)), ((), ())),
                               precision=lax.Precision.HIGHEST,
                               preferred_element_type=jnp.float32)

    csw_i = mm(wbins, tri_incl)
    csw_e = mm(wbins, tri_excl)
    csh_i = mm(hbins, tri_incl)
    csh_e = mm(hbins, tri_excl)
    cwk = 2.0 * BOUND * csw_e - BOUND
    cw_next = jnp.where(j == K - 1, BOUND, 2.0 * BOUND * csw_i - BOUND)
    chk = 2.0 * BOUND * csh_e - BOUND
    ch_next = jnp.where(j == K - 1, BOUND, 2.0 * BOUND * csh_i - BOUND)
    widths = cw_next - cwk
    heights = ch_next - chk
    delta = heights / widths

    lam = MIN_L + (1.0 - 2.0 * MIN_L) / (1.0 + jnp.exp(-ul_ref[...]))
    dk = jnp.where(j == 0, 1.0 - MIN_D, MIN_D + softplus(udl_ref[...]))
    dk1 = jnp.where(j == K - 1, 1.0 - MIN_D, MIN_D + softplus(udr_ref[...]))
    wb = jnp.sqrt(dk / dk1)
    wc = (lam * dk + (1.0 - lam) * wb * dk1) / delta
    ya = chk
    yb = ch_next
    yc = ((1.0 - lam) * ya + lam * wb * yb) / ((1.0 - lam) + lam * wb)

    A0 = -lam
    B0 = lam * ya
    a0 = wc - 1.0
    b0 = ya - wc * yc
    A1 = wc - lam * wb
    B1 = lam * wb * yb - wc * yc
    a1 = wc - wb
    b1 = wb * yb - wc * yc
    af0 = A0 * widths + cwk * a0
    bf0 = B0 * widths + cwk * b0
    af1 = A1 * widths + cwk * a1
    bf1 = B1 * widths + cwk * b1
    thr = jnp.where(j == K - 1, 1e30, ch_next + EPS)

    mm_ = means_ref[...]
    s10 = 10.0 * stds_ref[...]
    is_ = 1.0 / s10
    mis = mm_ * is_
    af0_ref[...] = af0 * is_
    bf0_ref[...] = bf0 - af0 * mis
    af1_ref[...] = af1 * is_
    bf1_ref[...] = bf1 - af1 * mis
    a0_ref[...] = a0 * is_
    b0_ref[...] = b0 - a0 * mis
    a1_ref[...] = a1 * is_
    b1_ref[...] = b1 - a1 * mis
    l0_ref[...] = jnp.log(wc * lam * (yc - ya) * widths)
    l1_ref[...] = jnp.log(wb * wc * (1.0 - lam) * (yb - yc) * widths)
    ycx = mm_ + yc * s10
    thrx = mm_ + thr * s10
    ycb = lax.bitcast_convert_type(ycx, jnp.int32)
    ycb = jnp.where(ycx >= 0, ycb + 1, ycb - 1)
    ycx = jnp.where(ycx == 0.0,
                    jnp.float32(1.1754944e-38),
                    lax.bitcast_convert_type(ycb, jnp.float32))
    yc_ref[...] = ycx
    thr_ref[...] = thrx
    is_ref[...] = is_
    clo = mm_ - BOUND * s10
    clo_ref[...] = clo
    csc_ref[...] = is_ * (G / 6.0)
    c0 = (-(D / 2) * jnp.log(2.0 * jnp.pi) - jnp.sum(jnp.log(s10)))
    c0_ref[...] = jnp.broadcast_to(c0, (1, 1))

    cellw = (6.0 / G) * s10
    base_i = lax.broadcasted_iota(jnp.int32, (D, GP), 1) * 4
    packed = jnp.zeros((D, GP), jnp.int32)
    for p in range(4):
        left = clo + (base_i + p).astype(jnp.float32) * cellw
        acc = jnp.zeros((D, GP), jnp.int32)
        for m in range(2 * K - 1):
            if m % 2 == 0:
                t = ycx[:, m // 2:m // 2 + 1]
            else:
                t = thrx[:, (m - 1) // 2:(m - 1) // 2 + 1]
            acc = acc + (left >= t).astype(jnp.int32)
        packed = packed | (jnp.minimum(acc, 2 * K - 4) << (8 * p))
    grid_ref[...] = packed


_DK = jax.ShapeDtypeStruct((D, K), jnp.float32)
_prep = pl.pallas_call(
    _prep_body,
    out_shape=[jax.ShapeDtypeStruct((D, 1), jnp.float32)] * 3 + [_DK] * 12
              + [jax.ShapeDtypeStruct((1, 1), jnp.float32),
                 jax.ShapeDtypeStruct((D, GP), jnp.int32)],
)


def _sc_body(x_hbm, tbl_hbm, grid_hbm, out_hbm, tbl_v, grid_v, x_v0, x_v1,
             out_v, sem0, sem1):
    wid = lax.axis_index("s") * NCORES + lax.axis_index("c")
    base = wid * SPW
    pltpu.sync_copy(tbl_hbm, tbl_v)
    pltpu.sync_copy(grid_hbm, grid_v)
    lane = lax.iota(jnp.int32, 16)
    c0v = tbl_v[pl.ds(OFF_C0, 16)]

    bufs = [x_v0, x_v1]
    sems = [sem0, sem1]
    nch = SPW // CH

    def start(c):
        return pltpu.async_copy(
            x_hbm.at[pl.ds((base + c * CH) * D, CH * D)], bufs[c % 2],
            sems[c % 2])

    pend = start(0)
    for chunk in range(nch):
        x_v = bufs[chunk % 2]
        pend.wait()
        if chunk + 1 < nch:
            pend = start(chunk + 1)

        def init_body(g, carry):
            out_v[pl.ds(chunk * CH + g * 16, 16)] = c0v
            return carry

        lax.fori_loop(0, NG, init_body, 0)

        def d_body(dd, carry):
            dvec = (dd + lane) & (D - 1)
            mv = plsc.load_gather(tbl_v, [dvec + OFF_M])
            isv = plsc.load_gather(tbl_v, [dvec + OFF_IS])
            clov = plsc.load_gather(tbl_v, [dvec + OFF_CLO])
            cscv = plsc.load_gather(tbl_v, [dvec + OFF_CSC])
            xib = lane * D + dvec
            rowv = dvec * ROW
            gbase = dvec * GROW
            tbase = rowv + OFF_T64
            caf = rowv + OFF_AF
            cbf = rowv + OFF_BF
            ca = rowv + OFF_A
            cb = rowv + OFF_B
            cl = rowv + OFF_L

            @plsc.parallel_loop(0, NG, step=1, unroll=4)
            def g_loop(g):
                xv = plsc.load_gather(x_v, [xib + g * (16 * D)])
                z1 = (xv - mv) * isv
                inside = (z1 >= -BOUND) & (z1 <= BOUND)
                cell = jnp.minimum(jnp.maximum(
                    ((xv - clov) * cscv).astype(jnp.int32), 0), G - 1)
                cw2 = plsc.load_gather(grid_v, [gbase + (cell >> 2)])
                lo = (cw2 >> ((cell & 3) << 3)) & 0xFF
                q = lo
                for i in range(3):
                    t = plsc.load_gather(tbl_v, [tbase + (lo + i)])
                    q = q + (xv >= t).astype(jnp.int32)
                af = plsc.load_gather(tbl_v, [q + caf])
                bf = plsc.load_gather(tbl_v, [q + cbf])
                av = plsc.load_gather(tbl_v, [q + ca])
                bv = plsc.load_gather(tbl_v, [q + cb])
                lv = plsc.load_gather(tbl_v, [q + cl])
                den = av * xv + bv
                z0 = (af * xv + bf) / den
                ad = jnp.abs(den)
                bits = plsc.bitcast(ad, jnp.int32)
                e = jnp.right_shift(bits, 23) - 127
                mant = plsc.bitcast(
                    (bits & 0x007FFFFF) | 0x3F800000, jnp.float32)
                t5 = mant - 1.0
                p = t5 * (0.99943129 + t5 * (-0.49130654 + t5 * (0.28768438
                         + t5 * (-0.13394622 + t5 * 0.03129158))))
                ln = e.astype(jnp.float32) * LN2 + p
                contrib = lv - 2.0 * ln - 0.5 * z0 * z0
                res = jnp.where(inside, contrib, -0.5 * z1 * z1)
                plsc.addupdate(out_v.at[pl.ds(chunk * CH + g * 16, 16)], res)

            return carry

        lax.fori_loop(0, D, d_body, 0)

    pltpu.sync_copy(out_v, out_hbm.at[pl.ds(base, SPW)])


_sc_spline = functools.partial(
    pl.kernel,
    out_type=jax.ShapeDtypeStruct((N,), jnp.float32),
    mesh=plsc.VectorSubcoreMesh(core_axis_name="c", subcore_axis_name="s"),
    compiler_params=pltpu.CompilerParams(needs_layout_passes=False),
    scratch_types=[
        pltpu.VMEM((TOTAL,), jnp.float32),
        pltpu.VMEM((D * GROW,), jnp.int32),
        pltpu.VMEM((CH * D,), jnp.float32),
        pltpu.VMEM((CH * D,), jnp.float32),
        pltpu.VMEM((SPW,), jnp.float32),
        pltpu.SemaphoreType.DMA,
        pltpu.SemaphoreType.DMA,
    ],
)(_sc_body)


def kernel(data_samples, ds_means, ds_stds, unnormalized_widths,
           unnormalized_heights, unnormalized_derivatives,
           unnormalized_lambdas):
    means2 = ds_means.reshape(D, 1)
    stds2 = ds_stds.reshape(D, 1)
    udl = jnp.pad(unnormalized_derivatives, ((0, 0), (1, 0)))
    udr = jnp.pad(unnormalized_derivatives, ((0, 0), (0, 1)))
    (is_, clo, csc, thr, yc, af0, af1, bf0, bf1, a0, a1, b0, b1, l0, l1, c0,
     grid) = _prep(means2, stds2, unnormalized_widths, unnormalized_heights,
                   udl, udr, unnormalized_lambdas)

    def il(x0, x1):
        x = jnp.stack([x0, x1], axis=2).reshape(D, 2 * K)
        return jnp.pad(x, ((0, 0), (0, 1))).reshape(-1)

    tbl = jnp.concatenate([
        ds_means.reshape(-1), is_.reshape(-1), clo.reshape(-1),
        csc.reshape(-1), il(yc, thr),
        il(af0, af1), il(bf0, bf1), il(a0, a1), il(b0, b1), il(l0, l1),
        jnp.broadcast_to(c0.reshape(()), (16,)),
    ])
    gridp = jnp.pad(grid, ((0, 0), (0, 1))).reshape(-1)
    return _sc_spline(data_samples.reshape(-1), tbl, gridp)


# speedup vs baseline: 1.1128x; 1.0214x over previous
import functools

import jax
import jax.numpy as jnp
from jax import lax
from jax.experimental import pallas as pl
from jax.experimental.pallas import tpu as pltpu
from jax.experimental.pallas import tpu_sc as plsc

N = 65536
D = 64
K = 32
BOUND = 3.0
MIN_BW = 1e-3
MIN_BH = 1e-3
MIN_D = 1e-3
MIN_L = 0.025
EPS = 1e-6
LN2 = 0.6931471805599453
G = 1024

ROW = 2 * K + 1
GP = G // 4
GROW = GP + 1
OFF_M = 0
OFF_IS = OFF_M + D
OFF_CLO = OFF_IS + D
OFF_CSC = OFF_CLO + D
OFF_T64 = OFF_CSC + D
OFF_AF = OFF_T64 + D * ROW
OFF_BF = OFF_AF + D * ROW
OFF_A = OFF_BF + D * ROW
OFF_B = OFF_A + D * ROW
OFF_L = OFF_B + D * ROW
OFF_C0 = OFF_L + D * ROW
TOTAL = OFF_C0 + 16

NCORES = 2
NSUB = 16
NW = NCORES * NSUB
SPW = N // NW
CH = 512
NG = CH // 16


def _prep_body(means_ref, stds_ref, uw_ref, uh_ref, udl_ref, udr_ref, ul_ref,
               is_ref, clo_ref, csc_ref, thr_ref, yc_ref,
               af0_ref, af1_ref, bf0_ref, bf1_ref,
               a0_ref, a1_ref, b0_ref, b1_ref, l0_ref, l1_ref, c0_ref,
               grid_ref):
    j = lax.broadcasted_iota(jnp.int32, (D, K), 1)
    kk = lax.broadcasted_iota(jnp.int32, (K, K), 0)
    jj = lax.broadcasted_iota(jnp.int32, (K, K), 1)
    tri_incl = (kk <= jj).astype(jnp.float32)
    tri_excl = (kk < jj).astype(jnp.float32)

    def softmax(x):
        e = jnp.exp(x - jnp.max(x, axis=-1, keepdims=True))
        return e / jnp.sum(e, axis=-1, keepdims=True)

    def softplus(x):
        return jnp.maximum(x, 0.0) + jnp.log(1.0 + jnp.exp(-jnp.abs(x)))

    w = softmax(uw_ref[...])
    h = softmax(uh_ref[...])
    wbins = MIN_BW + (1.0 - MIN_BW * K) * w
    hbins = MIN_BH + (1.0 - MIN_BH * K) * h

    def mm(x, t):
        return lax.dot_general(x, t, (((1,), (0,)), ((), ())),
                               precision=lax.Precision.HIGHEST,
                               preferred_element_type=jnp.float32)

    csw_i = mm(wbins, tri_incl)
    csw_e = mm(wbins, tri_excl)
    csh_i = mm(hbins, tri_incl)
    csh_e = mm(hbins, tri_excl)
    cwk = 2.0 * BOUND * csw_e - BOUND
    cw_next = jnp.where(j == K - 1, BOUND, 2.0 * BOUND * csw_i - BOUND)
    chk = 2.0 * BOUND * csh_e - BOUND
    ch_next = jnp.where(j == K - 1, BOUND, 2.0 * BOUND * csh_i - BOUND)
    widths = cw_next - cwk
    heights = ch_next - chk
    delta = heights / widths

    lam = MIN_L + (1.0 - 2.0 * MIN_L) / (1.0 + jnp.exp(-ul_ref[...]))
    dk = jnp.where(j == 0, 1.0 - MIN_D, MIN_D + softplus(udl_ref[...]))
    dk1 = jnp.where(j == K - 1, 1.0 - MIN_D, MIN_D + softplus(udr_ref[...]))
    wb = jnp.sqrt(dk / dk1)
    wc = (lam * dk + (1.0 - lam) * wb * dk1) / delta
    ya = chk
    yb = ch_next
    yc = ((1.0 - lam) * ya + lam * wb * yb) / ((1.0 - lam) + lam * wb)

    A0 = -lam
    B0 = lam * ya
    a0 = wc - 1.0
    b0 = ya - wc * yc
    A1 = wc - lam * wb
    B1 = lam * wb * yb - wc * yc
    a1 = wc - wb
    b1 = wb * yb - wc * yc
    af0 = A0 * widths + cwk * a0
    bf0 = B0 * widths + cwk * b0
    af1 = A1 * widths + cwk * a1
    bf1 = B1 * widths + cwk * b1
    thr = jnp.where(j == K - 1, 1e30, ch_next + EPS)

    mm_ = means_ref[...]
    s10 = 10.0 * stds_ref[...]
    is_ = 1.0 / s10
    mis = mm_ * is_
    af0_ref[...] = af0 * is_
    bf0_ref[...] = bf0 - af0 * mis
    af1_ref[...] = af1 * is_
    bf1_ref[...] = bf1 - af1 * mis
    a0_ref[...] = a0 * is_
    b0_ref[...] = b0 - a0 * mis
    a1_ref[...] = a1 * is_
    b1_ref[...] = b1 - a1 * mis
    l0_ref[...] = jnp.log(wc * lam * (yc - ya) * widths)
    l1_ref[...] = jnp.log(wb * wc * (1.0 - lam) * (yb - yc) * widths)
    ycx = mm_ + yc * s10
    thrx = mm_ + thr * s10
    ycb = lax.bitcast_convert_type(ycx, jnp.int32)
    ycb = jnp.where(ycx >= 0, ycb + 1, ycb - 1)
    ycx = jnp.where(ycx == 0.0,
                    jnp.float32(1.1754944e-38),
                    lax.bitcast_convert_type(ycb, jnp.float32))
    yc_ref[...] = ycx
    thr_ref[...] = thrx
    is_ref[...] = is_
    clo = mm_ - BOUND * s10
    clo_ref[...] = clo
    csc_ref[...] = is_ * (G / 6.0)
    c0 = (-(D / 2) * jnp.log(2.0 * jnp.pi) - jnp.sum(jnp.log(s10)))
    c0_ref[...] = jnp.broadcast_to(c0, (1, 1))

    cellw = (6.0 / G) * s10
    base_i = lax.broadcasted_iota(jnp.int32, (D, GP), 1) * 4
    packed = jnp.zeros((D, GP), jnp.int32)
    for p in range(4):
        left = clo + (base_i + p).astype(jnp.float32) * cellw
        acc = jnp.zeros((D, GP), jnp.int32)
        for m in range(2 * K - 1):
            if m % 2 == 0:
                t = ycx[:, m // 2:m // 2 + 1]
            else:
                t = thrx[:, (m - 1) // 2:(m - 1) // 2 + 1]
            acc = acc + (left >= t).astype(jnp.int32)
        packed = packed | (jnp.minimum(acc, 2 * K - 4) << (8 * p))
    grid_ref[...] = packed


_DK = jax.ShapeDtypeStruct((D, K), jnp.float32)
_prep = pl.pallas_call(
    _prep_body,
    out_shape=[jax.ShapeDtypeStruct((D, 1), jnp.float32)] * 3 + [_DK] * 12
              + [jax.ShapeDtypeStruct((1, 1), jnp.float32),
                 jax.ShapeDtypeStruct((D, GP), jnp.int32)],
)


def _sc_body(x_hbm, tbl_hbm, grid_hbm, out_hbm, tbl_v, grid_v, x_v0, x_v1,
             out_v, sem0, sem1):
    wid = lax.axis_index("s") * NCORES + lax.axis_index("c")
    base = wid * SPW
    pltpu.sync_copy(tbl_hbm, tbl_v)
    pltpu.sync_copy(grid_hbm, grid_v)
    lane = lax.iota(jnp.int32, 16)
    c0v = tbl_v[pl.ds(OFF_C0, 16)]

    bufs = [x_v0, x_v1]
    sems = [sem0, sem1]
    nch = SPW // CH

    def start(c):
        return pltpu.async_copy(
            x_hbm.at[pl.ds((base + c * CH) * D, CH * D)], bufs[c % 2],
            sems[c % 2])

    pend = start(0)
    for chunk in range(nch):
        x_v = bufs[chunk % 2]
        pend.wait()
        if chunk + 1 < nch:
            pend = start(chunk + 1)

        def init_body(g, carry):
            out_v[pl.ds(chunk * CH + g * 16, 16)] = c0v
            return carry

        lax.fori_loop(0, NG, init_body, 0)

        def d_body(dd, carry):
            dvec = (dd + lane) & (D - 1)
            mv = plsc.load_gather(tbl_v, [dvec + OFF_M])
            isv = plsc.load_gather(tbl_v, [dvec + OFF_IS])
            clov = plsc.load_gather(tbl_v, [dvec + OFF_CLO])
            cscv = plsc.load_gather(tbl_v, [dvec + OFF_CSC])
            xib = lane * D + dvec
            rowv = dvec * ROW
            gbase = dvec * GROW
            tbase = rowv + OFF_T64
            caf = rowv + OFF_AF
            cbf = rowv + OFF_BF
            ca = rowv + OFF_A
            cb = rowv + OFF_B
            cl = rowv + OFF_L

            @plsc.parallel_loop(0, NG, step=1, unroll=4)
            def g_loop(g):
                xv = plsc.load_gather(x_v, [xib + g * (16 * D)])
                z1 = (xv - mv) * isv
                inside = jnp.abs(z1) <= BOUND
                cell = jnp.minimum(jnp.maximum(
                    ((xv - clov) * cscv).astype(jnp.int32), 0), G - 1)
                cw2 = plsc.load_gather(grid_v, [gbase + (cell >> 2)])
                lo = (cw2 >> ((cell & 3) << 3)) & 0xFF
                q = lo
                for i in range(3):
                    t = plsc.load_gather(tbl_v, [tbase + (lo + i)])
                    q = q + (xv >= t).astype(jnp.int32)
                af = plsc.load_gather(tbl_v, [q + caf])
                bf = plsc.load_gather(tbl_v, [q + cbf])
                av = plsc.load_gather(tbl_v, [q + ca])
                bv = plsc.load_gather(tbl_v, [q + cb])
                lv = plsc.load_gather(tbl_v, [q + cl])
                den = av * xv + bv
                z0 = (af * xv + bf) / den
                ad = jnp.abs(den)
                bits = plsc.bitcast(ad, jnp.int32)
                e = jnp.right_shift(bits, 23) - 127
                mant = plsc.bitcast(
                    (bits & 0x007FFFFF) | 0x3F800000, jnp.float32)
                t5 = mant - 1.0
                p = t5 * (0.99943129 + t5 * (-0.49130654 + t5 * (0.28768438
                         + t5 * (-0.13394622 + t5 * 0.03129158))))
                ln = e.astype(jnp.float32) * LN2 + p
                v = jnp.where(inside, z0, z1)
                w = jnp.where(inside, lv - 2.0 * ln, 0.0)
                res = w - 0.5 * v * v
                plsc.addupdate(out_v.at[pl.ds(chunk * CH + g * 16, 16)], res)

            return carry

        lax.fori_loop(0, D, d_body, 0)

    pltpu.sync_copy(out_v, out_hbm.at[pl.ds(base, SPW)])


_sc_spline = functools.partial(
    pl.kernel,
    out_type=jax.ShapeDtypeStruct((N,), jnp.float32),
    mesh=plsc.VectorSubcoreMesh(core_axis_name="c", subcore_axis_name="s"),
    compiler_params=pltpu.CompilerParams(needs_layout_passes=False),
    scratch_types=[
        pltpu.VMEM((TOTAL,), jnp.float32),
        pltpu.VMEM((D * GROW,), jnp.int32),
        pltpu.VMEM((CH * D,), jnp.float32),
        pltpu.VMEM((CH * D,), jnp.float32),
        pltpu.VMEM((SPW,), jnp.float32),
        pltpu.SemaphoreType.DMA,
        pltpu.SemaphoreType.DMA,
    ],
)(_sc_body)


def kernel(data_samples, ds_means, ds_stds, unnormalized_widths,
           unnormalized_heights, unnormalized_derivatives,
           unnormalized_lambdas):
    means2 = ds_means.reshape(D, 1)
    stds2 = ds_stds.reshape(D, 1)
    udl = jnp.pad(unnormalized_derivatives, ((0, 0), (1, 0)))
    udr = jnp.pad(unnormalized_derivatives, ((0, 0), (0, 1)))
    (is_, clo, csc, thr, yc, af0, af1, bf0, bf1, a0, a1, b0, b1, l0, l1, c0,
     grid) = _prep(means2, stds2, unnormalized_widths, unnormalized_heights,
                   udl, udr, unnormalized_lambdas)

    def il(x0, x1):
        x = jnp.stack([x0, x1], axis=2).reshape(D, 2 * K)
        return jnp.pad(x, ((0, 0), (0, 1))).reshape(-1)

    tbl = jnp.concatenate([
        ds_means.reshape(-1), is_.reshape(-1), clo.reshape(-1),
        csc.reshape(-1), il(yc, thr),
        il(af0, af1), il(bf0, bf1), il(a0, a1), il(b0, b1), il(l0, l1),
        jnp.broadcast_to(c0.reshape(()), (16,)),
    ])
    gridp = jnp.pad(grid, ((0, 0), (0, 1))).reshape(-1)
    return _sc_spline(data_samples.reshape(-1), tbl, gridp)
